# Initial kernel scaffold; baseline (speedup 1.0000x reference)
import jax
import jax.numpy as jnp
from jax.experimental import pallas as pl


def kernel(x, adj, W1, b1, W2, b2, Wf, bf):
    raise NotImplementedError("write your pallas kernel here")


# speedup vs baseline: 1.0124x; 1.0124x over previous
import functools

import jax
import jax.numpy as jnp
from jax.experimental import pallas as pl
from jax.experimental.pallas import tpu as pltpu

_BLK = 400


def _layer1_body(adj_ref, x_ref, w1_ref, b1_ref, out_ref):
    t = jnp.dot(adj_ref[...], x_ref[...], preferred_element_type=jnp.float32)
    h = jnp.dot(t, w1_ref[...], preferred_element_type=jnp.float32) + b1_ref[...]
    out_ref[...] = jnp.maximum(h, 0.0)


def _layer2_body(adj_ref, h1_ref, w2_ref, b2_ref, wf_ref, bf_ref, out_ref):
    t = jnp.dot(adj_ref[...], h1_ref[...], preferred_element_type=jnp.float32)
    h = jnp.maximum(
        jnp.dot(t, w2_ref[...], preferred_element_type=jnp.float32) + b2_ref[...],
        0.0,
    )
    logits = jnp.dot(h, wf_ref[...], preferred_element_type=jnp.float32) + bf_ref[...]
    m = jnp.max(logits, axis=1, keepdims=True)
    lse = jnp.log(jnp.sum(jnp.exp(logits - m), axis=1, keepdims=True))
    out_ref[...] = logits - m - lse


@jax.jit
def kernel(x, adj, W1, b1, W2, b2, Wf, bf):
    n, f = x.shape
    h1dim = W1.shape[1]
    h2dim = W2.shape[1]
    c = Wf.shape[1]
    blk = _BLK
    grid = (n // blk,)

    adj_spec = pl.BlockSpec((blk, n), lambda i: (i, 0))
    full = lambda shape: pl.BlockSpec(shape, lambda i: (0,) * len(shape))

    h1 = pl.pallas_call(
        _layer1_body,
        grid=grid,
        in_specs=[adj_spec, full((n, f)), full((f, h1dim)), full((1, h1dim))],
        out_specs=pl.BlockSpec((blk, h1dim), lambda i: (i, 0)),
        out_shape=jax.ShapeDtypeStruct((n, h1dim), jnp.float32),
    )(adj, x, W1, b1.reshape(1, -1))

    out = pl.pallas_call(
        _layer2_body,
        grid=grid,
        in_specs=[
            adj_spec,
            full((n, h1dim)),
            full((h1dim, h2dim)),
            full((1, h2dim)),
            full((h2dim, c)),
            full((1, c)),
        ],
        out_specs=pl.BlockSpec((blk, c), lambda i: (i, 0)),
        out_shape=jax.ShapeDtypeStruct((n, c), jnp.float32),
    )(adj, h1, W2, b2.reshape(1, -1), Wf, bf.reshape(1, -1))

    return out
